# baseline (speedup 1.0000x reference)
import functools

import jax
import jax.numpy as jnp
from jax import lax
from jax.experimental import pallas as pl
from jax.experimental.pallas import tpu as pltpu
from jax.experimental.pallas import tpu_sc as plsc

_EMBED = 64
_DM = 128
_HID = 256
_STEPS = 4
_NW = 32
_CHUNK = 25
_NBUF = 3


def _sc_segment_sum(emb, idx, total_rows, rows_per_w, n_chunks):
  mesh = plsc.VectorSubcoreMesh(core_axis_name="c", subcore_axis_name="s")
  ids_per_row = n_chunks * _CHUNK

  del total_rows
  @functools.partial(
      pl.kernel,
      out_type=jax.ShapeDtypeStruct((_NW, rows_per_w, _DM), jnp.float32),
      mesh=mesh,
      scratch_types=[
          pltpu.VMEM((rows_per_w, n_chunks, _CHUNK), jnp.int32),
          pltpu.VMEM((_NBUF, ids_per_row, _EMBED), jnp.float32),
          pltpu.VMEM((rows_per_w, _DM), jnp.float32),
          pltpu.SemaphoreType.DMA((_NBUF,)),
      ],
      compiler_params=pltpu.CompilerParams(use_tc_tiling_on_sc=False),
  )
  def seg_sum(emb_hbm, idx_hbm, out_hbm, idx_v, buf_v, out_v, sem):
    wid = lax.axis_index("s") * 2 + lax.axis_index("c")
    pltpu.sync_copy(idx_hbm.at[wid], idx_v)

    def stream_descs(r, p):
      return [
          pltpu.make_async_copy(
              emb_hbm.at[idx_v.at[r, k]],
              buf_v.at[p, pl.ds(k * _CHUNK, _CHUNK)],
              sem.at[p],
          )
          for k in range(n_chunks)
      ]

    for r0 in range(_NBUF - 1):
      for cp in stream_descs(r0, r0):
        cp.start()

    def row_body(r, carry):
      p = lax.rem(r, _NBUF)
      @pl.when(r + _NBUF - 1 < rows_per_w)
      def _():
        for cp in stream_descs(r + _NBUF - 1, lax.rem(r + _NBUF - 1, _NBUF)):
          cp.start()

      for cp in stream_descs(r, p):
        cp.wait()

      def acc_body(j, acc):
        acc = list(acc)
        for u in range(4):
          row = 8 * j + 2 * u
          for v in range(4):
            acc[v] = acc[v] + buf_v[p, row, pl.ds(v * 16, 16)]
          for v in range(4):
            acc[4 + v] = acc[4 + v] + buf_v[p, row + 1, pl.ds(v * 16, 16)]
        return tuple(acc)

      zero = jnp.zeros((16,), jnp.float32)
      acc = lax.fori_loop(0, ids_per_row // 8, acc_body, (zero,) * 8)
      for v in range(8):
        out_v[r, pl.ds(v * 16, 16)] = acc[v]
      return carry

    lax.fori_loop(0, rows_per_w, row_body, 0)
    pltpu.sync_copy(out_v, out_hbm.at[wid])

  return seg_sum(emb, idx)


def _dense_body(qls, qrs, sls, srs, qld, qrd, sld, srd, gcn_w, gcn_b,
                p1w, p1b, p2w, p2b, ln_a, ln_b, w_ih, w_hh, b_ih, b_hh,
                nbr_ref, out):
  f32 = jnp.float32
  nbr = nbr_ref[0, 0]
  gw = gcn_w[...]
  gb = gcn_b[...]

  def enc(s, d):
    y = lax.dot_general(s, gw, (((1,), (1,)), ((), ())),
                        preferred_element_type=f32)
    y = (y + nbr * gb) / d
    return jnp.tanh(y)

  ql = enc(qls[...], qld[...])
  qr = enc(qrs[...], qrd[...])
  sl = enc(sls[...], sld[...])
  sr = enc(srs[...], srd[...])
  qn = jnp.concatenate([ql, qr], axis=1)
  sn = jnp.concatenate([sl, sr], axis=1)

  p1 = p1w[...]
  p2 = p2w[...]
  la = ln_a[...]
  lb = ln_b[...]

  def sup(x):
    h = jnp.maximum(
        lax.dot_general(x, p1, (((1,), (1,)), ((), ())),
                        preferred_element_type=f32) + p1b[...], 0.0)
    h = lax.dot_general(h, p2, (((1,), (1,)), ((), ())),
                        preferred_element_type=f32) + p2b[...]
    z = h + x
    mu = jnp.mean(z, axis=1, keepdims=True)
    var = jnp.sum((z - mu) ** 2, axis=1, keepdims=True) / (z.shape[1] - 1)
    return (z - mu) / (jnp.sqrt(var) + 1e-3) * la + lb

  few = 5
  sg = jnp.mean(sup(sn)[0:few], axis=0, keepdims=True)
  qg = sup(qn)

  bsz = qg.shape[0]
  wih = w_ih[...]
  whh = w_hh[...]
  gi = lax.dot_general(qg, wih, (((1,), (1,)), ((), ())),
                       preferred_element_type=f32) + b_ih[...]
  rr = jnp.broadcast_to(sg, (bsz, _DM))
  c = jnp.zeros((bsz, _HID), f32)
  hr = jnp.zeros((bsz, _HID), f32)
  h = qg
  for _ in range(_STEPS):
    gates = gi + lax.dot_general(hr, whh, (((1,), (1,)), ((), ())),
                                 preferred_element_type=f32) + b_hh[...]
    ig = jax.nn.sigmoid(gates[:, 0:_HID])
    fg = jax.nn.sigmoid(gates[:, _HID:2 * _HID])
    gg = jnp.tanh(gates[:, 2 * _HID:3 * _HID])
    og = jax.nn.sigmoid(gates[:, 3 * _HID:4 * _HID])
    c = fg * c + ig * gg
    hn = og * jnp.tanh(c)
    h = qg + hn[:, 0:_DM]
    hr = jnp.concatenate([h, rr], axis=1)
  out[...] = jnp.sum(h * sg, axis=1, keepdims=True)


def kernel(query, support, query_left_connections, query_left_degrees,
           query_right_connections, query_right_degrees,
           support_left_connections, support_left_degrees,
           support_right_connections, support_right_degrees,
           symbol_emb, gcn_w_w, gcn_w_b, proj1_w, proj1_b, proj2_w, proj2_b,
           ln_a, ln_b, w_ih, w_hh, b_ih, b_hh):
  bsz, nbr = query_left_connections.shape[0], query_left_connections.shape[1]
  few = support_left_connections.shape[0]
  ids_per_row = 2 * nbr
  n_chunks = ids_per_row // _CHUNK
  total = 2 * bsz + 2 * few
  rows_per_w = -(-total // _NW)
  padded = rows_per_w * _NW

  i32 = jnp.int32
  ids = jnp.concatenate([
      query_left_connections.reshape(bsz, ids_per_row).astype(i32),
      query_right_connections.reshape(bsz, ids_per_row).astype(i32),
      support_left_connections.reshape(few, ids_per_row).astype(i32),
      support_right_connections.reshape(few, ids_per_row).astype(i32),
      jnp.zeros((padded - total, ids_per_row), i32),
  ], axis=0)
  idx = ids.reshape(_NW, rows_per_w, n_chunks, _CHUNK)

  sums = _sc_segment_sum(symbol_emb.astype(jnp.float32), idx,
                         padded, rows_per_w, n_chunks)
  sums = sums.reshape(padded, _DM)

  f32 = jnp.float32
  pad3 = jnp.zeros((8 - few, _DM), f32)
  qls = sums[0:bsz]
  qrs = sums[bsz:2 * bsz]
  sls = jnp.concatenate([sums[2 * bsz:2 * bsz + few], pad3], axis=0)
  srs = jnp.concatenate([sums[2 * bsz + few:2 * bsz + 2 * few], pad3], axis=0)

  one3 = jnp.ones((8 - few, 1), f32)
  qld = query_left_degrees.reshape(bsz, 1).astype(f32)
  qrd = query_right_degrees.reshape(bsz, 1).astype(f32)
  sld = jnp.concatenate([support_left_degrees.reshape(few, 1).astype(f32), one3], axis=0)
  srd = jnp.concatenate([support_right_degrees.reshape(few, 1).astype(f32), one3], axis=0)

  nbr_arr = jnp.full((1, 1), float(nbr), f32)

  scores = pl.pallas_call(
      _dense_body,
      out_shape=jax.ShapeDtypeStruct((bsz, 1), f32),
  )(qls, qrs, sls, srs, qld, qrd, sld, srd,
    gcn_w_w.astype(f32), gcn_w_b.reshape(1, -1).astype(f32),
    proj1_w.astype(f32), proj1_b.reshape(1, -1).astype(f32),
    proj2_w.astype(f32), proj2_b.reshape(1, -1).astype(f32),
    ln_a.reshape(1, -1).astype(f32), ln_b.reshape(1, -1).astype(f32),
    w_ih.astype(f32), w_hh.astype(f32),
    b_ih.reshape(1, -1).astype(f32), b_hh.reshape(1, -1).astype(f32),
    nbr_arr)
  return scores[:, 0]


# speedup vs baseline: 1.3996x; 1.3996x over previous
import functools

import jax
import jax.numpy as jnp
from jax import lax
from jax.experimental import pallas as pl
from jax.experimental.pallas import tpu as pltpu
from jax.experimental.pallas import tpu_sc as plsc

_EMBED = 64
_DM = 128
_HID = 256
_STEPS = 4
_NW = 32
_CHUNK = 25
_NBUF = 3


def _sc_segment_sum(emb, idx, total_rows, rows_per_w, n_chunks):
  mesh = plsc.VectorSubcoreMesh(core_axis_name="c", subcore_axis_name="s")
  ids_per_row = n_chunks * _CHUNK

  del total_rows
  @functools.partial(
      pl.kernel,
      out_type=jax.ShapeDtypeStruct((_NW, rows_per_w, _DM), jnp.float32),
      mesh=mesh,
      scratch_types=[
          pltpu.VMEM((rows_per_w, n_chunks, _CHUNK), jnp.int32),
          pltpu.VMEM((_NBUF, ids_per_row, _EMBED), jnp.float32),
          pltpu.VMEM((rows_per_w, _DM), jnp.float32),
          pltpu.SemaphoreType.DMA((_NBUF,)),
      ],
      compiler_params=pltpu.CompilerParams(use_tc_tiling_on_sc=False),
  )
  def seg_sum(emb_hbm, idx_hbm, out_hbm, idx_v, buf_v, out_v, sem):
    wid = lax.axis_index("s") * 2 + lax.axis_index("c")
    pltpu.sync_copy(idx_hbm.at[wid], idx_v)

    def stream_descs(r, p):
      return [
          pltpu.make_async_copy(
              emb_hbm.at[idx_v.at[r, k]],
              buf_v.at[p, pl.ds(k * _CHUNK, _CHUNK)],
              sem.at[p],
          )
          for k in range(n_chunks)
      ]

    for r0 in range(_NBUF - 1):
      for cp in stream_descs(r0, r0):
        cp.start()

    def row_body(r, carry):
      p = lax.rem(r, _NBUF)
      @pl.when(r + _NBUF - 1 < rows_per_w)
      def _():
        for cp in stream_descs(r + _NBUF - 1, lax.rem(r + _NBUF - 1, _NBUF)):
          cp.start()

      for cp in stream_descs(r, p):
        cp.wait()

      def acc_body(j, acc):
        acc = list(acc)
        for u in range(4):
          row = 8 * j + 2 * u
          for v in range(4):
            acc[v] = acc[v] + buf_v[p, row, pl.ds(v * 16, 16)]
          for v in range(4):
            acc[4 + v] = acc[4 + v] + buf_v[p, row + 1, pl.ds(v * 16, 16)]
        return tuple(acc)

      zero = jnp.zeros((16,), jnp.float32)
      acc = lax.fori_loop(0, ids_per_row // 8, acc_body, (zero,) * 8)
      for v in range(8):
        out_v[r, pl.ds(v * 16, 16)] = acc[v]
      return carry

    lax.fori_loop(0, rows_per_w, row_body, 0)
    pltpu.sync_copy(out_v, out_hbm.at[wid])

  return seg_sum(emb, idx)


def _dense_body(qls, qrs, sls, srs, qld, qrd, sld, srd, gcn_w, gcn_b,
                p1w, p1b, p2w, p2b, ln_a, ln_b, w_ih, w_hh, b_ih, b_hh,
                nbr_ref, out):
  f32 = jnp.float32
  nbr = nbr_ref[0, 0]
  gw = gcn_w[...]
  gb = gcn_b[...]

  def enc(s, d):
    y = lax.dot_general(s, gw, (((1,), (1,)), ((), ())),
                        preferred_element_type=f32)
    y = (y + nbr * gb) / d
    return jnp.tanh(y)

  ql = enc(qls[...], qld[...])
  qr = enc(qrs[...], qrd[...])
  sl = enc(sls[...], sld[...])
  sr = enc(srs[...], srd[...])
  qn = jnp.concatenate([ql, qr], axis=1)
  sn = jnp.concatenate([sl, sr], axis=1)

  p1 = p1w[...]
  p2 = p2w[...]
  la = ln_a[...]
  lb = ln_b[...]

  def sup(x):
    h = jnp.maximum(
        lax.dot_general(x, p1, (((1,), (1,)), ((), ())),
                        preferred_element_type=f32) + p1b[...], 0.0)
    h = lax.dot_general(h, p2, (((1,), (1,)), ((), ())),
                        preferred_element_type=f32) + p2b[...]
    z = h + x
    mu = jnp.mean(z, axis=1, keepdims=True)
    var = jnp.sum((z - mu) ** 2, axis=1, keepdims=True) / (z.shape[1] - 1)
    return (z - mu) / (jnp.sqrt(var) + 1e-3) * la + lb

  few = 5
  sg = jnp.mean(sup(sn)[0:few], axis=0, keepdims=True)
  qg = sup(qn)

  bsz = qg.shape[0]
  wih = w_ih[...]
  whh = w_hh[...]
  gi = lax.dot_general(qg, wih, (((1,), (1,)), ((), ())),
                       preferred_element_type=f32) + b_ih[...]
  rr = jnp.broadcast_to(sg, (bsz, _DM))
  c = jnp.zeros((bsz, _HID), f32)
  hr = jnp.zeros((bsz, _HID), f32)
  h = qg
  for _ in range(_STEPS):
    gates = gi + lax.dot_general(hr, whh, (((1,), (1,)), ((), ())),
                                 preferred_element_type=f32) + b_hh[...]
    ig = jax.nn.sigmoid(gates[:, 0:_HID])
    fg = jax.nn.sigmoid(gates[:, _HID:2 * _HID])
    gg = jnp.tanh(gates[:, 2 * _HID:3 * _HID])
    og = jax.nn.sigmoid(gates[:, 3 * _HID:4 * _HID])
    c = fg * c + ig * gg
    hn = og * jnp.tanh(c)
    h = qg + hn[:, 0:_DM]
    hr = jnp.concatenate([h, rr], axis=1)
  out[...] = jnp.sum(h * sg, axis=1, keepdims=True)


def kernel(query, support, query_left_connections, query_left_degrees,
           query_right_connections, query_right_degrees,
           support_left_connections, support_left_degrees,
           support_right_connections, support_right_degrees,
           symbol_emb, gcn_w_w, gcn_w_b, proj1_w, proj1_b, proj2_w, proj2_b,
           ln_a, ln_b, w_ih, w_hh, b_ih, b_hh):
  bsz, nbr = query_left_connections.shape[0], query_left_connections.shape[1]
  few = support_left_connections.shape[0]
  ids_per_row = 2 * nbr
  n_chunks = ids_per_row // _CHUNK
  total = 2 * bsz + 2 * few
  rows_per_w = -(-total // _NW)
  padded = rows_per_w * _NW

  i32 = jnp.int32
  ids = jnp.concatenate([
      query_left_connections.reshape(bsz, ids_per_row).astype(i32),
      query_right_connections.reshape(bsz, ids_per_row).astype(i32),
      support_left_connections.reshape(few, ids_per_row).astype(i32),
      support_right_connections.reshape(few, ids_per_row).astype(i32),
      jnp.broadcast_to(
          jax.lax.iota(i32, ids_per_row)[None, :] * 997,
          (padded - total, ids_per_row)),
  ], axis=0)
  idx = ids.reshape(_NW, rows_per_w, n_chunks, _CHUNK)

  sums = _sc_segment_sum(symbol_emb.astype(jnp.float32), idx,
                         padded, rows_per_w, n_chunks)
  sums = sums.reshape(padded, _DM)

  f32 = jnp.float32
  pad3 = jnp.zeros((8 - few, _DM), f32)
  qls = sums[0:bsz]
  qrs = sums[bsz:2 * bsz]
  sls = jnp.concatenate([sums[2 * bsz:2 * bsz + few], pad3], axis=0)
  srs = jnp.concatenate([sums[2 * bsz + few:2 * bsz + 2 * few], pad3], axis=0)

  one3 = jnp.ones((8 - few, 1), f32)
  qld = query_left_degrees.reshape(bsz, 1).astype(f32)
  qrd = query_right_degrees.reshape(bsz, 1).astype(f32)
  sld = jnp.concatenate([support_left_degrees.reshape(few, 1).astype(f32), one3], axis=0)
  srd = jnp.concatenate([support_right_degrees.reshape(few, 1).astype(f32), one3], axis=0)

  nbr_arr = jnp.full((1, 1), float(nbr), f32)

  scores = pl.pallas_call(
      _dense_body,
      out_shape=jax.ShapeDtypeStruct((bsz, 1), f32),
  )(qls, qrs, sls, srs, qld, qrd, sld, srd,
    gcn_w_w.astype(f32), gcn_w_b.reshape(1, -1).astype(f32),
    proj1_w.astype(f32), proj1_b.reshape(1, -1).astype(f32),
    proj2_w.astype(f32), proj2_b.reshape(1, -1).astype(f32),
    ln_a.reshape(1, -1).astype(f32), ln_b.reshape(1, -1).astype(f32),
    w_ih.astype(f32), w_hh.astype(f32),
    b_ih.reshape(1, -1).astype(f32), b_hh.reshape(1, -1).astype(f32),
    nbr_arr)
  return scores[:, 0]
